# baseline (speedup 1.0000x reference)
import functools

import jax
import jax.numpy as jnp
from jax import lax
from jax.experimental import pallas as pl
from jax.experimental.pallas import tpu as pltpu
from jax.experimental.pallas import tpu_sc as plsc

B = 16384
D = 32
L = 16
NCOLS = 7813
COLS_PER_W = 248
CHUNK_COLS = 4
CHUNK_V = CHUNK_COLS * 128
NCHUNK = COLS_PER_W // CHUNK_COLS
MAXCOL0 = NCOLS - CHUNK_COLS
LISTCAP = 1024
CLCAP = 128
IDXCHUNK = 1024
QROWS = 256


@functools.cache
def _build_gather(nw, nc):
    mesh = plsc.VectorSubcoreMesh(core_axis_name="c", subcore_axis_name="s")
    row_t = jax.ShapeDtypeStruct((B, D), jnp.float32)

    @functools.partial(
        pl.kernel,
        out_type=(row_t, row_t),
        mesh=mesh,
        compiler_params=pltpu.CompilerParams(
            needs_layout_passes=False, disable_bounds_checks=True),
        scratch_types=[
            pltpu.VMEM((IDXCHUNK,), jnp.int32),
            pltpu.VMEM((LISTCAP + L,), jnp.int32),
            pltpu.VMEM((LISTCAP + L,), jnp.int32),
            pltpu.VMEM((LISTCAP + L,), jnp.int32),
            pltpu.VMEM((LISTCAP + L,), jnp.int32),
            pltpu.VMEM((D, CHUNK_V), jnp.float32),
            pltpu.VMEM((D, CHUNK_V), jnp.float32),
            pltpu.VMEM((D, CHUNK_V), jnp.float32),
            pltpu.VMEM((D, CHUNK_V), jnp.float32),
            pltpu.VMEM((LISTCAP + L,), jnp.int32),
            pltpu.VMEM((LISTCAP + L,), jnp.int32),
            pltpu.VMEM((LISTCAP + L,), jnp.int32),
            pltpu.VMEM((LISTCAP + L,), jnp.int32),
            pltpu.VMEM((CLCAP, D), jnp.float32),
            pltpu.VMEM((NCHUNK + 2 + L,), jnp.int32),
            pltpu.VMEM((NCHUNK + 2 + L,), jnp.int32),
            pltpu.VMEM((NCHUNK + 2 + L,), jnp.int32),
            pltpu.VMEM((NCHUNK + 2 + L,), jnp.int32),
            pltpu.SMEM((NCHUNK + 2 + L,), jnp.int32),
            pltpu.SemaphoreType.DMA,
            pltpu.SemaphoreType.DMA,
            pltpu.SemaphoreType.DMA,
        ],
    )
    def gather(c_hbm, t_hbm, cemb_hbm, temb_hbm, rowsc_hbm, rowst_hbm,
               idxbuf, lcc, lcb, ltc, ltb, chunkc, chunkt, chunkc2, chunkt2,
               scc, scb, stc, stb, stage, cntc, cntt, offsc, offst,
               cursor, gsema, gsemb, ssem):
        wid = lax.axis_index("s") * nc + lax.axis_index("c")
        lo = wid * COLS_PER_W
        iota = lax.iota(jnp.int32, L)

        def build_list(idx_hbm, list_c, list_b):
            def outer(j, n):
                pltpu.sync_copy(idx_hbm.at[pl.ds(j * IDXCHUNK, IDXCHUNK)],
                                idxbuf)

                def inner(k, n):
                    c = idxbuf[pl.ds(k * L, L)]
                    bvec = j * IDXCHUNK + k * L + iota
                    colv = lax.shift_right_logical(c, 7)
                    m = (colv >= lo) & (colv < lo + COLS_PER_W)
                    off = jnp.minimum(n, LISTCAP - L)
                    plsc.store_compressed(list_c.at[pl.ds(off, L)], c, mask=m)
                    plsc.store_compressed(list_b.at[pl.ds(off, L)], bvec,
                                          mask=m)
                    return n + plsc.all_reduce_population_count(m)[0]

                return lax.fori_loop(0, IDXCHUNK // L, inner, n)

            return lax.fori_loop(0, B // IDXCHUNK, outer, 0)

        n_c = jnp.minimum(build_list(c_hbm, lcc, lcb), LISTCAP)
        n_t = jnp.minimum(build_list(t_hbm, ltc, ltb), LISTCAP)

        def build_hist(list_c, n, cnt):
            zero = jnp.zeros((L,), jnp.int32)
            for z in range((NCHUNK + 2 + L) // L):
                cnt[pl.ds(z * L, L)] = zero

            def histstep(k, carry):
                c = list_c[pl.ds(k * L, L)]
                ch = lax.shift_right_logical(
                    lax.shift_right_logical(c, 7) - lo, 2)
                ch = jnp.clip(ch, 0, NCHUNK - 1)
                valid = (k * L + iota) < n
                plsc.addupdate_scatter(cnt, [ch],
                                       jnp.ones((L,), jnp.int32), mask=valid)
                return carry

            lax.fori_loop(0, (n + L - 1) // L, histstep, 0)

        build_hist(lcc, n_c, cntc)
        build_hist(ltc, n_t, cntt)

        def chunk_sort(list_c, list_b, n, cnt, offs, out_c, out_b):
            running = 0
            for z in range((NCHUNK + 2 + L) // L):
                v = cnt[pl.ds(z * L, L)]
                cs = plsc.cumsum(v)
                excl = cs - v + running
                offs[pl.ds(z * L, L)] = excl
                for q in range(L):
                    cursor[z * L + q] = excl[q]
                running = running + cs[L - 1]

            def place(e, carry):
                cval = list_c[pl.ds(e, L)][0]
                bval = list_b[pl.ds(e, L)][0]
                ch = lax.shift_right_logical(
                    lax.shift_right_logical(cval, 7) - lo, 2)
                ch = jnp.clip(ch, 0, NCHUNK - 1)
                pos = cursor[ch]
                cursor[ch] = pos + 1
                v1 = out_c[pl.ds(pos, L)]
                out_c[pl.ds(pos, L)] = jnp.where(iota == 0, cval, v1)
                v2 = out_b[pl.ds(pos, L)]
                out_b[pl.ds(pos, L)] = jnp.where(iota == 0, bval, v2)
                return carry

            lax.fori_loop(0, n, place, 0)

        chunk_sort(lcc, lcb, n_c, cntc, offsc, scc, scb)
        chunk_sort(ltc, ltb, n_t, cntt, offst, stc, stb)

        def cnt_at(cnt, k):
            return cnt[pl.ds(k, L)][0]

        def process(chunk, sort_c, sort_b, offs, cnt, rows_hbm, col0, k):
            start = offs[pl.ds(k, L)][0]
            m = jnp.minimum(cnt_at(cnt, k), CLCAP)

            def ext(e, carry):
                cval = sort_c[pl.ds(start + e, L)][0]
                bval = sort_b[pl.ds(start + e, L)][0]
                q = jnp.zeros((L,), jnp.int32) + (cval - col0 * 128)
                lo16 = plsc.load_gather(chunk, [iota, q])
                hi16 = plsc.load_gather(chunk, [iota + L, q])
                stage[e, pl.ds(0, L)] = lo16
                stage[e, pl.ds(L, L)] = hi16
                pltpu.async_copy(stage.at[e], rows_hbm.at[bval], ssem)
                return carry

            lax.fori_loop(0, m, ext, 0)

            def drain(e, carry):
                pltpu.make_async_copy(stage.at[0], rows_hbm.at[0],
                                      ssem).wait()
                return carry

            lax.fori_loop(0, m, drain, 0)

        def vstart(k):
            col0 = jnp.minimum(lo + k * CHUNK_COLS, MAXCOL0)
            return pl.multiple_of(col0 * 128, 128), col0

        def issue_chunk(k, cbuf, tbuf, sem):
            v0, _ = vstart(k)

            @pl.when(cnt_at(cntc, k) > 0)
            def _():
                pltpu.async_copy(cemb_hbm.at[:, pl.ds(v0, CHUNK_V)], cbuf,
                                 sem)

            @pl.when(cnt_at(cntt, k) > 0)
            def _():
                pltpu.async_copy(temb_hbm.at[:, pl.ds(v0, CHUNK_V)], tbuf,
                                 sem)

        def wait_process(k, cbuf, tbuf, sem):
            v0, col0 = vstart(k)

            @pl.when(cnt_at(cntc, k) > 0)
            def _():
                pltpu.make_async_copy(cemb_hbm.at[:, pl.ds(v0, CHUNK_V)],
                                      cbuf, sem).wait()
                process(cbuf, scc, scb, offsc, cntc, rowsc_hbm, col0, k)

            @pl.when(cnt_at(cntt, k) > 0)
            def _():
                pltpu.make_async_copy(temb_hbm.at[:, pl.ds(v0, CHUNK_V)],
                                      tbuf, sem).wait()
                process(tbuf, stc, stb, offst, cntt, rowst_hbm, col0, k)

        issue_chunk(0, chunkc, chunkt, gsema)

        def chunk_pair(j, carry):
            k0 = j * 2
            issue_chunk(k0 + 1, chunkc2, chunkt2, gsemb)
            wait_process(k0, chunkc, chunkt, gsema)

            @pl.when(k0 + 2 < NCHUNK)
            def _():
                issue_chunk(k0 + 2, chunkc, chunkt, gsema)

            wait_process(k0 + 1, chunkc2, chunkt2, gsemb)
            return carry

        lax.fori_loop(0, NCHUNK // 2, chunk_pair, 0)

    return gather


@functools.cache
def _build_dot(nw, nc, bpw):
    ngroups = bpw // L
    mesh = plsc.VectorSubcoreMesh(core_axis_name="c", subcore_axis_name="s")

    @functools.partial(
        pl.kernel,
        out_type=jax.ShapeDtypeStruct((B,), jnp.float32),
        mesh=mesh,
        compiler_params=pltpu.CompilerParams(needs_layout_passes=False),
        scratch_types=[
            pltpu.VMEM((QROWS, D), jnp.float32),
            pltpu.VMEM((QROWS, D), jnp.float32),
            pltpu.VMEM((QROWS,), jnp.float32),
        ],
    )
    def dot(rowsc_hbm, rowst_hbm, out_hbm, crow, trow, scores):
        wid = lax.axis_index("s") * nc + lax.axis_index("c")
        base = wid * bpw
        iota = lax.iota(jnp.int32, L)

        def qstep(q, carry):
            qb = base + q * QROWS
            pltpu.sync_copy(rowsc_hbm.at[pl.ds(qb, QROWS)], crow)
            pltpu.sync_copy(rowst_hbm.at[pl.ds(qb, QROWS)], trow)

            def group(g, carry2):
                acc = jnp.zeros((L,), jnp.float32)
                for i in range(L):
                    b = g * L + i
                    c_lo = crow[b, pl.ds(0, L)]
                    c_hi = crow[b, pl.ds(L, L)]
                    t_lo = trow[b, pl.ds(0, L)]
                    t_hi = trow[b, pl.ds(L, L)]
                    s = jnp.sum(c_lo * t_lo + c_hi * t_hi)
                    acc = jnp.where(iota == i, s, acc)
                scores[pl.ds(g * L, L)] = acc
                return carry2

            lax.fori_loop(0, QROWS // L, group, 0)
            pltpu.sync_copy(scores, out_hbm.at[pl.ds(qb, QROWS)])
            return carry

        lax.fori_loop(0, bpw // QROWS, qstep, 0)

    return dot


def kernel(center_idx, target_idx, in_emb, out_emb):
    info = plsc.get_sparse_core_info()
    nw = info.num_cores * info.num_subcores
    g = _build_gather(nw, info.num_cores)
    d = _build_dot(nw, info.num_cores, B // nw)
    rowsc, rowst = g(center_idx, target_idx, in_emb.T, out_emb.T)
    return d(rowsc, rowst)


# speedup vs baseline: 1.1454x; 1.1454x over previous
import functools

import jax
import jax.numpy as jnp
from jax import lax
from jax.experimental import pallas as pl
from jax.experimental.pallas import tpu as pltpu
from jax.experimental.pallas import tpu_sc as plsc

B = 16384
D = 32
L = 16
NCOLS = 7813
COLS_PER_W = 248
CHUNK_COLS = 4
CHUNK_V = CHUNK_COLS * 128
NCHUNK = COLS_PER_W // CHUNK_COLS
MAXCOL0 = NCOLS - CHUNK_COLS
LISTCAP = 1024
CLCAP = 128
IDXCHUNK = 1024
QROWS = 256


@functools.cache
def _build_gather(nw, nc):
    mesh = plsc.VectorSubcoreMesh(core_axis_name="c", subcore_axis_name="s")
    row_t = jax.ShapeDtypeStruct((B, D), jnp.float32)

    @functools.partial(
        pl.kernel,
        out_type=(row_t, row_t),
        mesh=mesh,
        compiler_params=pltpu.CompilerParams(
            needs_layout_passes=False, disable_bounds_checks=True),
        scratch_types=[
            pltpu.VMEM((IDXCHUNK,), jnp.int32),
            pltpu.VMEM((LISTCAP + L,), jnp.int32),
            pltpu.VMEM((LISTCAP + L,), jnp.int32),
            pltpu.VMEM((LISTCAP + L,), jnp.int32),
            pltpu.VMEM((LISTCAP + L,), jnp.int32),
            pltpu.VMEM((D, CHUNK_V), jnp.float32),
            pltpu.VMEM((D, CHUNK_V), jnp.float32),
            pltpu.VMEM((D, CHUNK_V), jnp.float32),
            pltpu.VMEM((D, CHUNK_V), jnp.float32),
            pltpu.VMEM((CLCAP + L,), jnp.int32),
            pltpu.VMEM((CLCAP + L,), jnp.int32),
            pltpu.VMEM((CLCAP, D), jnp.float32),
            pltpu.VMEM((NCHUNK + 2 + L,), jnp.int32),
            pltpu.VMEM((NCHUNK + 2 + L,), jnp.int32),
            pltpu.SemaphoreType.DMA,
            pltpu.SemaphoreType.DMA,
            pltpu.SemaphoreType.DMA,
        ],
    )
    def gather(c_hbm, t_hbm, cemb_hbm, temb_hbm, rowsc_hbm, rowst_hbm,
               idxbuf, lcc, lcb, ltc, ltb, chunkc, chunkt, chunkc2, chunkt2,
               clc, clb, stage, cntc, cntt, gsema, gsemb, ssem):
        wid = lax.axis_index("s") * nc + lax.axis_index("c")
        lo = wid * COLS_PER_W
        iota = lax.iota(jnp.int32, L)

        def build_list(idx_hbm, list_c, list_b):
            def outer(j, n):
                pltpu.sync_copy(idx_hbm.at[pl.ds(j * IDXCHUNK, IDXCHUNK)],
                                idxbuf)

                def inner(k, n):
                    c = idxbuf[pl.ds(k * L, L)]
                    bvec = j * IDXCHUNK + k * L + iota
                    colv = lax.shift_right_logical(c, 7)
                    m = (colv >= lo) & (colv < lo + COLS_PER_W)
                    off = jnp.minimum(n, LISTCAP - L)
                    plsc.store_compressed(list_c.at[pl.ds(off, L)], c, mask=m)
                    plsc.store_compressed(list_b.at[pl.ds(off, L)], bvec,
                                          mask=m)
                    return n + plsc.all_reduce_population_count(m)[0]

                return lax.fori_loop(0, IDXCHUNK // L, inner, n)

            return lax.fori_loop(0, B // IDXCHUNK, outer, 0)

        n_c = jnp.minimum(build_list(c_hbm, lcc, lcb), LISTCAP)
        n_t = jnp.minimum(build_list(t_hbm, ltc, ltb), LISTCAP)

        def build_hist(list_c, n, cnt):
            zero = jnp.zeros((L,), jnp.int32)
            for z in range((NCHUNK + 2 + L) // L):
                cnt[pl.ds(z * L, L)] = zero

            def histstep(k, carry):
                c = list_c[pl.ds(k * L, L)]
                ch = lax.shift_right_logical(
                    lax.shift_right_logical(c, 7) - lo, 2)
                ch = jnp.clip(ch, 0, NCHUNK - 1)
                valid = (k * L + iota) < n
                plsc.addupdate_scatter(cnt, [ch],
                                       jnp.ones((L,), jnp.int32), mask=valid)
                return carry

            lax.fori_loop(0, (n + L - 1) // L, histstep, 0)

        build_hist(lcc, n_c, cntc)
        build_hist(ltc, n_t, cntt)

        def cnt_at(cnt, k):
            return cnt[pl.ds(k, L)][0]

        def process(chunk, list_c, list_b, n, rows_hbm, col0):
            def filt(k, m):
                c = list_c[pl.ds(k * L, L)]
                b = list_b[pl.ds(k * L, L)]
                colv = lax.shift_right_logical(c, 7)
                msk = ((colv >= col0) & (colv < col0 + CHUNK_COLS)
                       & ((k * L + iota) < n))
                off = jnp.minimum(m, CLCAP - L)
                plsc.store_compressed(clc.at[pl.ds(off, L)], c, mask=msk)
                plsc.store_compressed(clb.at[pl.ds(off, L)], b, mask=msk)
                return m + plsc.all_reduce_population_count(msk)[0]

            m = lax.fori_loop(0, (n + L - 1) // L, filt, 0)
            m = jnp.minimum(m, CLCAP)

            def ext(e, carry):
                cval = clc[pl.ds(e, L)][0]
                bval = clb[pl.ds(e, L)][0]
                q = jnp.zeros((L,), jnp.int32) + (cval - col0 * 128)
                lo16 = plsc.load_gather(chunk, [iota, q])
                hi16 = plsc.load_gather(chunk, [iota + L, q])
                stage[e, pl.ds(0, L)] = lo16
                stage[e, pl.ds(L, L)] = hi16
                pltpu.async_copy(stage.at[e], rows_hbm.at[bval], ssem)
                return carry

            lax.fori_loop(0, m, ext, 0)

            def drain(e, carry):
                pltpu.make_async_copy(stage.at[0], rows_hbm.at[0],
                                      ssem).wait()
                return carry

            lax.fori_loop(0, m, drain, 0)

        def vstart(k):
            col0 = jnp.minimum(lo + k * CHUNK_COLS, MAXCOL0)
            return pl.multiple_of(col0 * 128, 128), col0

        def issue_chunk(k, cbuf, tbuf, sem):
            v0, _ = vstart(k)

            @pl.when(cnt_at(cntc, k) > 0)
            def _():
                pltpu.async_copy(cemb_hbm.at[:, pl.ds(v0, CHUNK_V)], cbuf,
                                 sem)

            @pl.when(cnt_at(cntt, k) > 0)
            def _():
                pltpu.async_copy(temb_hbm.at[:, pl.ds(v0, CHUNK_V)], tbuf,
                                 sem)

        def wait_process(k, cbuf, tbuf, sem):
            v0, col0 = vstart(k)

            @pl.when(cnt_at(cntc, k) > 0)
            def _():
                pltpu.make_async_copy(cemb_hbm.at[:, pl.ds(v0, CHUNK_V)],
                                      cbuf, sem).wait()
                process(cbuf, lcc, lcb, n_c, rowsc_hbm, col0)

            @pl.when(cnt_at(cntt, k) > 0)
            def _():
                pltpu.make_async_copy(temb_hbm.at[:, pl.ds(v0, CHUNK_V)],
                                      tbuf, sem).wait()
                process(tbuf, ltc, ltb, n_t, rowst_hbm, col0)

        issue_chunk(0, chunkc, chunkt, gsema)

        def chunk_pair(j, carry):
            k0 = j * 2
            issue_chunk(k0 + 1, chunkc2, chunkt2, gsemb)
            wait_process(k0, chunkc, chunkt, gsema)

            @pl.when(k0 + 2 < NCHUNK)
            def _():
                issue_chunk(k0 + 2, chunkc, chunkt, gsema)

            wait_process(k0 + 1, chunkc2, chunkt2, gsemb)
            return carry

        lax.fori_loop(0, NCHUNK // 2, chunk_pair, 0)

    return gather


@functools.cache
def _build_dot(nw, nc, bpw):
    ngroups = bpw // L
    mesh = plsc.VectorSubcoreMesh(core_axis_name="c", subcore_axis_name="s")

    @functools.partial(
        pl.kernel,
        out_type=jax.ShapeDtypeStruct((B,), jnp.float32),
        mesh=mesh,
        compiler_params=pltpu.CompilerParams(needs_layout_passes=False),
        scratch_types=[
            pltpu.VMEM((QROWS, D), jnp.float32),
            pltpu.VMEM((QROWS, D), jnp.float32),
            pltpu.VMEM((QROWS,), jnp.float32),
        ],
    )
    def dot(rowsc_hbm, rowst_hbm, out_hbm, crow, trow, scores):
        wid = lax.axis_index("s") * nc + lax.axis_index("c")
        base = wid * bpw
        iota = lax.iota(jnp.int32, L)

        def qstep(q, carry):
            qb = base + q * QROWS
            pltpu.sync_copy(rowsc_hbm.at[pl.ds(qb, QROWS)], crow)
            pltpu.sync_copy(rowst_hbm.at[pl.ds(qb, QROWS)], trow)

            def group(g, carry2):
                acc = jnp.zeros((L,), jnp.float32)
                for i in range(L):
                    b = g * L + i
                    c_lo = crow[b, pl.ds(0, L)]
                    c_hi = crow[b, pl.ds(L, L)]
                    t_lo = trow[b, pl.ds(0, L)]
                    t_hi = trow[b, pl.ds(L, L)]
                    s = jnp.sum(c_lo * t_lo + c_hi * t_hi)
                    acc = jnp.where(iota == i, s, acc)
                scores[pl.ds(g * L, L)] = acc
                return carry2

            lax.fori_loop(0, QROWS // L, group, 0)
            pltpu.sync_copy(scores, out_hbm.at[pl.ds(qb, QROWS)])
            return carry

        lax.fori_loop(0, bpw // QROWS, qstep, 0)

    return dot


def kernel(center_idx, target_idx, in_emb, out_emb):
    info = plsc.get_sparse_core_info()
    nw = info.num_cores * info.num_subcores
    g = _build_gather(nw, info.num_cores)
    d = _build_dot(nw, info.num_cores, B // nw)
    rowsc, rowst = g(center_idx, target_idx, in_emb.T, out_emb.T)
    return d(rowsc, rowst)
